# baseline (speedup 1.0000x reference)
import functools

import jax
import jax.numpy as jnp
from jax import lax
from jax.experimental import pallas as pl
from jax.experimental.pallas import tpu as pltpu
from jax.experimental.pallas import tpu_sc as plsc

N_NODES = 100000
N_NEIGH = 64
L = 16
NC, NS = 2, 16
NW = NC * NS
CH = 256

NSC = 40960
NFULL = NSC // CH
MAXC = NFULL // NW
NPAIR = (MAXC + 1) // 2
NTC = N_NODES - NSC
BT = 8192


def _sc_body(
    x_hbm, w_hbm, b_hbm, o_hbm,
    xv0, wv0, bv0, ov0, xv1, wv1, bv1, ov1,
    sin0, sin1, sout0, sout1,
):
    wid = lax.axis_index("s") * NC + lax.axis_index("c")
    xvs, wvs, bvs, ovs = (xv0, xv1), (wv0, wv1), (bv0, bv1), (ov0, ov1)
    sins, souts = (sin0, sin1), (sout0, sout1)

    def start_in(k, b):
        base = k * CH
        pltpu.async_copy(x_hbm.at[:, pl.ds(base, CH)], xvs[b], sins[b])
        pltpu.async_copy(w_hbm.at[:, pl.ds(base, CH)], wvs[b], sins[b])
        pltpu.async_copy(b_hbm.at[pl.ds(base, CH)], bvs[b], sins[b])

    def wait_in(k, b):
        base = k * CH
        pltpu.make_async_copy(x_hbm.at[:, pl.ds(base, CH)], xvs[b], sins[b]).wait()
        pltpu.make_async_copy(w_hbm.at[:, pl.ds(base, CH)], wvs[b], sins[b]).wait()
        pltpu.make_async_copy(b_hbm.at[pl.ds(base, CH)], bvs[b], sins[b]).wait()

    def start_out(k, b):
        pltpu.async_copy(ovs[b], o_hbm.at[pl.ds(k * CH, CH)], souts[b])

    def wait_out(k, b):
        pltpu.make_async_copy(ovs[b], o_hbm.at[pl.ds(k * CH, CH)], souts[b]).wait()

    def compute(xr, wr, br, orr):
        def group_iter(it, c2):
            sl = pl.ds(it * L, L)
            accs = [xr[j, sl] * wr[j, sl] for j in range(4)]
            for j in range(4, N_NEIGH):
                accs[j % 4] = accs[j % 4] + xr[j, sl] * wr[j, sl]
            orr[sl] = br[sl] + ((accs[0] + accs[1]) + (accs[2] + accs[3]))
            return c2

        lax.fori_loop(0, CH // L, group_iter, 0)

    start_in(wid, 0)

    def pair_body(p, carry):
        i0 = 2 * p
        k0 = wid + i0 * NW
        k1 = k0 + NW

        @pl.when(i0 + 1 < MAXC)
        def _():
            start_in(k1, 1)

        wait_in(k0, 0)

        @pl.when(i0 >= 2)
        def _():
            wait_out(k0 - 2 * NW, 0)

        compute(xvs[0], wvs[0], bvs[0], ovs[0])
        start_out(k0, 0)

        @pl.when(i0 + 2 < MAXC)
        def _():
            start_in(k1 + NW, 0)

        @pl.when(i0 + 1 < MAXC)
        def _():
            wait_in(k1, 1)

            @pl.when(i0 >= 1)
            def _():
                wait_out(k1 - 2 * NW, 1)

            compute(xvs[1], wvs[1], bvs[1], ovs[1])
            start_out(k1, 1)

        return carry

    lax.fori_loop(0, NPAIR, pair_body, 0)

    ilast = MAXC - 1
    klast = wid + ilast * NW
    if ilast % 2 == 0:
        wait_out(klast, 0)
        if MAXC >= 2:
            wait_out(klast - NW, 1)
    else:
        wait_out(klast, 1)
        if MAXC >= 2:
            wait_out(klast - NW, 0)


def _tc_body(x_ref, w_ref, b_ref, o_ref):
    o_ref[...] = b_ref[...] + jnp.sum(x_ref[...] * w_ref[...], axis=0)


@jax.jit
def kernel(layer_input, weight, bias):
    xT = layer_input.T
    wT = weight.T

    mesh = plsc.VectorSubcoreMesh(core_axis_name="c", subcore_axis_name="s")
    sc_run = pl.kernel(
        _sc_body,
        out_type=jax.ShapeDtypeStruct((NSC,), jnp.float32),
        mesh=mesh,
        compiler_params=pltpu.CompilerParams(
            needs_layout_passes=False,
        ),
        scratch_types=[
            pltpu.VMEM((N_NEIGH, CH), jnp.float32),
            pltpu.VMEM((N_NEIGH, CH), jnp.float32),
            pltpu.VMEM((CH,), jnp.float32),
            pltpu.VMEM((CH,), jnp.float32),
            pltpu.VMEM((N_NEIGH, CH), jnp.float32),
            pltpu.VMEM((N_NEIGH, CH), jnp.float32),
            pltpu.VMEM((CH,), jnp.float32),
            pltpu.VMEM((CH,), jnp.float32),
            pltpu.SemaphoreType.DMA,
            pltpu.SemaphoreType.DMA,
            pltpu.SemaphoreType.DMA,
            pltpu.SemaphoreType.DMA,
        ],
    )
    sc_out = sc_run(xT, wT, bias)

    off = NSC // BT
    tc_out = pl.pallas_call(
        _tc_body,
        grid=(pl.cdiv(NTC, BT),),
        in_specs=[
            pl.BlockSpec((N_NEIGH, BT), lambda i: (0, off + i)),
            pl.BlockSpec((N_NEIGH, BT), lambda i: (0, off + i)),
            pl.BlockSpec((BT,), lambda i: (off + i,)),
        ],
        out_specs=pl.BlockSpec((BT,), lambda i: (i,)),
        out_shape=jax.ShapeDtypeStruct((NTC,), jnp.float32),
    )(xT, wT, bias)

    return jnp.concatenate([sc_out, tc_out])


# speedup vs baseline: 1.1332x; 1.1332x over previous
import functools

import jax
import jax.numpy as jnp
from jax import lax
from jax.experimental import pallas as pl
from jax.experimental.pallas import tpu as pltpu
from jax.experimental.pallas import tpu_sc as plsc

N_NODES = 100000
N_NEIGH = 64
L = 16
NC, NS = 2, 16
NW = NC * NS
CH = 256

NSC = 32768
NFULL = NSC // CH
MAXC = NFULL // NW
NPAIR = (MAXC + 1) // 2
NTC = N_NODES - NSC
BT = 16384


def _sc_body(
    x_hbm, w_hbm, b_hbm, o_hbm,
    xv0, wv0, bv0, ov0, xv1, wv1, bv1, ov1,
    sin0, sin1, sout0, sout1,
):
    wid = lax.axis_index("s") * NC + lax.axis_index("c")
    xvs, wvs, bvs, ovs = (xv0, xv1), (wv0, wv1), (bv0, bv1), (ov0, ov1)
    sins, souts = (sin0, sin1), (sout0, sout1)

    def start_in(k, b):
        base = k * CH
        pltpu.async_copy(x_hbm.at[:, pl.ds(base, CH)], xvs[b], sins[b])
        pltpu.async_copy(w_hbm.at[:, pl.ds(base, CH)], wvs[b], sins[b])
        pltpu.async_copy(b_hbm.at[pl.ds(base, CH)], bvs[b], sins[b])

    def wait_in(k, b):
        base = k * CH
        pltpu.make_async_copy(x_hbm.at[:, pl.ds(base, CH)], xvs[b], sins[b]).wait()
        pltpu.make_async_copy(w_hbm.at[:, pl.ds(base, CH)], wvs[b], sins[b]).wait()
        pltpu.make_async_copy(b_hbm.at[pl.ds(base, CH)], bvs[b], sins[b]).wait()

    def start_out(k, b):
        pltpu.async_copy(ovs[b], o_hbm.at[pl.ds(k * CH, CH)], souts[b])

    def wait_out(k, b):
        pltpu.make_async_copy(ovs[b], o_hbm.at[pl.ds(k * CH, CH)], souts[b]).wait()

    def compute(xr, wr, br, orr):
        def group_iter(it, c2):
            sl = pl.ds(it * L, L)

            def col_block(jb, accs):
                j0 = jb * L
                a = list(accs)
                for m in range(L):
                    a[m % 4] = a[m % 4] + xr[j0 + m, sl] * wr[j0 + m, sl]
                return tuple(a)

            z = jnp.zeros((L,), jnp.float32)
            a0, a1, a2, a3 = lax.fori_loop(
                0, N_NEIGH // L, col_block, (z, z, z, z)
            )
            orr[sl] = br[sl] + ((a0 + a1) + (a2 + a3))
            return c2

        lax.fori_loop(0, CH // L, group_iter, 0)

    start_in(wid, 0)

    def pair_body(p, carry):
        i0 = 2 * p
        k0 = wid + i0 * NW
        k1 = k0 + NW

        @pl.when(i0 + 1 < MAXC)
        def _():
            start_in(k1, 1)

        wait_in(k0, 0)

        @pl.when(i0 >= 2)
        def _():
            wait_out(k0 - 2 * NW, 0)

        compute(xvs[0], wvs[0], bvs[0], ovs[0])
        start_out(k0, 0)

        @pl.when(i0 + 2 < MAXC)
        def _():
            start_in(k1 + NW, 0)

        @pl.when(i0 + 1 < MAXC)
        def _():
            wait_in(k1, 1)

            @pl.when(i0 >= 1)
            def _():
                wait_out(k1 - 2 * NW, 1)

            compute(xvs[1], wvs[1], bvs[1], ovs[1])
            start_out(k1, 1)

        return carry

    lax.fori_loop(0, NPAIR, pair_body, 0)

    ilast = MAXC - 1
    klast = wid + ilast * NW
    if ilast % 2 == 0:
        wait_out(klast, 0)
        if MAXC >= 2:
            wait_out(klast - NW, 1)
    else:
        wait_out(klast, 1)
        if MAXC >= 2:
            wait_out(klast - NW, 0)


def _tc_body(x_ref, w_ref, b_ref, o_ref):
    o_ref[...] = b_ref[...] + jnp.sum(x_ref[...] * w_ref[...], axis=0)


@jax.jit
def kernel(layer_input, weight, bias):
    xT = layer_input.T
    wT = weight.T

    mesh = plsc.VectorSubcoreMesh(core_axis_name="c", subcore_axis_name="s")
    sc_run = pl.kernel(
        _sc_body,
        out_type=jax.ShapeDtypeStruct((NSC,), jnp.float32),
        mesh=mesh,
        compiler_params=pltpu.CompilerParams(
            needs_layout_passes=False,
        ),
        scratch_types=[
            pltpu.VMEM((N_NEIGH, CH), jnp.float32),
            pltpu.VMEM((N_NEIGH, CH), jnp.float32),
            pltpu.VMEM((CH,), jnp.float32),
            pltpu.VMEM((CH,), jnp.float32),
            pltpu.VMEM((N_NEIGH, CH), jnp.float32),
            pltpu.VMEM((N_NEIGH, CH), jnp.float32),
            pltpu.VMEM((CH,), jnp.float32),
            pltpu.VMEM((CH,), jnp.float32),
            pltpu.SemaphoreType.DMA,
            pltpu.SemaphoreType.DMA,
            pltpu.SemaphoreType.DMA,
            pltpu.SemaphoreType.DMA,
        ],
    )
    sc_out = sc_run(xT, wT, bias)

    off = NSC // BT
    tc_out = pl.pallas_call(
        _tc_body,
        grid=(pl.cdiv(NTC, BT),),
        in_specs=[
            pl.BlockSpec((N_NEIGH, BT), lambda i: (0, off + i)),
            pl.BlockSpec((N_NEIGH, BT), lambda i: (0, off + i)),
            pl.BlockSpec((BT,), lambda i: (off + i,)),
        ],
        out_specs=pl.BlockSpec((BT,), lambda i: (i,)),
        out_shape=jax.ShapeDtypeStruct((NTC,), jnp.float32),
    )(xT, wT, bias)

    return jnp.concatenate([sc_out, tc_out])
